# baseline (speedup 1.0000x reference)
import functools

import jax
import jax.numpy as jnp
from jax import lax
from jax.experimental import pallas as pl
from jax.experimental.pallas import tpu as pltpu
from jax.experimental.pallas import tpu_sc as plsc

BATCH = 4096
SEQ = 200
SEQ_PAD = 256
EMBED_DIM = 32

NUM_CORES = 2
NUM_SUBCORES = 16
NUM_WORKERS = NUM_CORES * NUM_SUBCORES
ROWS_PER_W = BATCH // NUM_WORKERS
IDX_PER_W = ROWS_PER_W * SEQ_PAD

_mesh = plsc.VectorSubcoreMesh(core_axis_name="c", subcore_axis_name="s")


@functools.partial(
    pl.kernel,
    mesh=_mesh,
    out_type=jax.ShapeDtypeStruct((BATCH, SEQ, EMBED_DIM), jnp.float32),
    scratch_types=[
        pltpu.VMEM((IDX_PER_W,), jnp.int32),
        pltpu.VMEM((2, SEQ, EMBED_DIM), jnp.float32),
        pltpu.SemaphoreType.DMA,
        pltpu.SemaphoreType.DMA,
    ],
    compiler_params=pltpu.CompilerParams(use_tc_tiling_on_sc=False),
)
def _emb_lookup(table_hbm, idx_hbm, out_hbm, idx_v, rows_v, sem_g, sem_o):
    wid = lax.axis_index("s") * NUM_CORES + lax.axis_index("c")
    row0 = wid * ROWS_PER_W
    pltpu.sync_copy(idx_hbm.at[pl.ds(wid * IDX_PER_W, IDX_PER_W)], idx_v)

    def start_gather(r, buf):
        off = pl.multiple_of(r * SEQ_PAD, 8)
        pltpu.async_copy(
            table_hbm.at[idx_v.at[pl.ds(off, SEQ)]], rows_v.at[buf], sem_g
        )

    def wait_gather(buf):
        pltpu.make_async_copy(
            table_hbm.at[idx_v.at[pl.ds(0, SEQ)]], rows_v.at[buf], sem_g
        ).wait()

    def start_out(r, buf):
        pltpu.async_copy(rows_v.at[buf], out_hbm.at[row0 + r], sem_o)

    def wait_out(buf):
        pltpu.make_async_copy(rows_v.at[buf], out_hbm.at[row0], sem_o).wait()

    start_gather(0, 0)

    def body(r, carry):
        buf = lax.rem(r, 2)
        wait_gather(buf)
        start_out(r, buf)

        @pl.when(r + 1 < ROWS_PER_W)
        def _():
            @pl.when(r >= 1)
            def _():
                wait_out(1 - buf)

            start_gather(r + 1, 1 - buf)

        return carry

    lax.fori_loop(0, ROWS_PER_W, body, 0)
    wait_out(0)
    wait_out(1)


def kernel(x, weight):
    idx_flat = jnp.pad(x.astype(jnp.int32), ((0, 0), (0, SEQ_PAD - SEQ)))
    idx_flat = idx_flat.reshape(-1)
    return _emb_lookup(weight, idx_flat)
